# Initial kernel scaffold; baseline (speedup 1.0000x reference)
import jax
import jax.numpy as jnp
from jax.experimental import pallas as pl


def kernel(x, W_enc, b_enc, W_dec, b_dec):
    raise NotImplementedError("write your pallas kernel here")


# speedup vs baseline: 11.0343x; 11.0343x over previous
import functools

import jax
import jax.numpy as jnp
from jax.experimental import pallas as pl
from jax.experimental.pallas import tpu as pltpu

_K = 64


def _enc_body(x_ref, w_ref, b_ref, o_ref):
    acc = jax.lax.dot_general(
        x_ref[...].astype(jnp.bfloat16), w_ref[...].astype(jnp.bfloat16),
        (((1,), (1,)), ((), ())),
        preferred_element_type=jnp.float32,
    )
    o_ref[...] = jnp.maximum(acc + b_ref[...], 0.0)


def _encode(x, W_enc, b_enc, *, br=512, bd=1024):
    B, D_IN = x.shape
    D_DICT = W_enc.shape[0]
    grid = (B // br, D_DICT // bd)
    return pl.pallas_call(
        _enc_body,
        grid=grid,
        in_specs=[
            pl.BlockSpec((br, D_IN), lambda i, j: (i, 0)),
            pl.BlockSpec((bd, D_IN), lambda i, j: (j, 0)),
            pl.BlockSpec((1, bd), lambda i, j: (0, j)),
        ],
        out_specs=pl.BlockSpec((br, bd), lambda i, j: (i, j)),
        out_shape=jax.ShapeDtypeStruct((B, D_DICT), jnp.float32),
    )(x, W_enc, b_enc.reshape(1, D_DICT))


def _topk_body(pre_ref, lat_ref, *, n_iter=31):
    pre = pre_ref[...]
    bits = pltpu.bitcast(pre, jnp.int32)
    rows = pre.shape[0]
    lo = jnp.zeros((rows, 1), jnp.int32)
    hi = jnp.full((rows, 1), 0x7F800000, jnp.int32)

    def body(_, carry):
        lo, hi = carry
        mid = lo + jax.lax.shift_right_logical(hi - lo, 1)
        cnt = jnp.sum((bits >= mid).astype(jnp.int32), axis=1, keepdims=True)
        ok = cnt >= _K
        return jnp.where(ok, mid, lo), jnp.where(ok, hi, mid)

    lo, hi = jax.lax.fori_loop(0, n_iter, body, (lo, hi))
    lat_ref[...] = jnp.where(bits >= lo, pre, 0.0)


def _topk_mask(pre, *, br=128):
    B, D_DICT = pre.shape
    return pl.pallas_call(
        _topk_body,
        grid=(B // br,),
        in_specs=[pl.BlockSpec((br, D_DICT), lambda i: (i, 0))],
        out_specs=pl.BlockSpec((br, D_DICT), lambda i: (i, 0)),
        out_shape=jax.ShapeDtypeStruct((B, D_DICT), jnp.float32),
    )(pre)


def _dec_body(lat_ref, w_ref, b_ref, o_ref):
    k = pl.program_id(1)

    @pl.when(k == 0)
    def _():
        o_ref[...] = jnp.broadcast_to(b_ref[...], o_ref.shape)

    lat = lat_ref[...].astype(jnp.bfloat16)
    o_ref[...] += jax.lax.dot_general(
        lat, w_ref[...],
        (((1,), (1,)), ((), ())),
        preferred_element_type=jnp.float32,
    )


def _decode(latents, W_dec_bf16, b_dec, *, br=512, bk=2048):
    B, D_DICT = latents.shape
    D_IN = W_dec_bf16.shape[0]
    grid = (B // br, D_DICT // bk)
    return pl.pallas_call(
        _dec_body,
        grid=grid,
        in_specs=[
            pl.BlockSpec((br, bk), lambda i, k: (i, k)),
            pl.BlockSpec((D_IN, bk), lambda i, k: (0, k)),
            pl.BlockSpec((1, D_IN), lambda i, k: (0, 0)),
        ],
        out_specs=pl.BlockSpec((br, D_IN), lambda i, k: (i, 0)),
        out_shape=jax.ShapeDtypeStruct((B, D_IN), jnp.float32),
    )(latents, W_dec_bf16, b_dec.reshape(1, D_IN))


def kernel(x, W_enc, b_enc, W_dec, b_dec):
    pre = _encode(x, W_enc, b_enc)
    latents = _topk_mask(pre)
    x_hat = _decode(latents, W_dec.astype(jnp.bfloat16), b_dec)
    return (x_hat, latents)


# speedup vs baseline: 11.5761x; 1.0491x over previous
import jax
import jax.numpy as jnp
from jax.experimental import pallas as pl
from jax.experimental.pallas import tpu as pltpu

_K = 64


def _enc_body(x_ref, w_ref, b_ref, o_ref):
    acc = jax.lax.dot_general(
        x_ref[...], w_ref[...],
        (((1,), (1,)), ((), ())),
        preferred_element_type=jnp.float32,
    )
    o_ref[...] = jnp.maximum(acc + b_ref[...], 0.0)


def _encode(x_bf16, W_enc_bf16, b_enc, *, br=512, bd=1024):
    B, D_IN = x_bf16.shape
    D_DICT = W_enc_bf16.shape[0]
    grid = (D_DICT // bd, B // br)
    return pl.pallas_call(
        _enc_body,
        grid=grid,
        in_specs=[
            pl.BlockSpec((br, D_IN), lambda j, i: (i, 0)),
            pl.BlockSpec((bd, D_IN), lambda j, i: (j, 0)),
            pl.BlockSpec((1, bd), lambda j, i: (0, j)),
        ],
        out_specs=pl.BlockSpec((br, bd), lambda j, i: (i, j)),
        out_shape=jax.ShapeDtypeStruct((B, D_DICT), jnp.float32),
    )(x_bf16, W_enc_bf16, b_enc.reshape(1, D_DICT))


def _topk_body(pre_ref, lat_ref, *, n_iter=31):
    pre = pre_ref[...]
    bits = pltpu.bitcast(pre, jnp.int32)
    rows = pre.shape[0]
    lo = jnp.zeros((rows, 1), jnp.int32)
    hi = jnp.full((rows, 1), 0x7F800000, jnp.int32)

    def body(_, carry):
        lo, hi = carry
        mid = lo + jax.lax.shift_right_logical(hi - lo, 1)
        cnt = jnp.sum((bits >= mid).astype(jnp.int32), axis=1, keepdims=True)
        ok = cnt >= _K
        return jnp.where(ok, mid, lo), jnp.where(ok, hi, mid)

    lo, hi = jax.lax.fori_loop(0, n_iter, body, (lo, hi))
    lat_ref[...] = jnp.where(bits >= lo, pre, 0.0)


def _topk_mask(pre, *, br=128):
    B, D_DICT = pre.shape
    return pl.pallas_call(
        _topk_body,
        grid=(B // br,),
        in_specs=[pl.BlockSpec((br, D_DICT), lambda i: (i, 0))],
        out_specs=pl.BlockSpec((br, D_DICT), lambda i: (i, 0)),
        out_shape=jax.ShapeDtypeStruct((B, D_DICT), jnp.float32),
    )(pre)


def _dec_body(lat_ref, w_ref, b_ref, o_ref):
    k = pl.program_id(1)

    @pl.when(k == 0)
    def _():
        o_ref[...] = jnp.broadcast_to(b_ref[...], o_ref.shape)

    lat = lat_ref[...].astype(jnp.bfloat16)
    o_ref[...] += jax.lax.dot_general(
        lat, w_ref[...],
        (((1,), (1,)), ((), ())),
        preferred_element_type=jnp.float32,
    )


def _decode(latents, W_dec_bf16, b_dec, *, br=1024, bk=2048):
    B, D_DICT = latents.shape
    D_IN = W_dec_bf16.shape[0]
    grid = (B // br, D_DICT // bk)
    return pl.pallas_call(
        _dec_body,
        grid=grid,
        in_specs=[
            pl.BlockSpec((br, bk), lambda i, k: (i, k)),
            pl.BlockSpec((D_IN, bk), lambda i, k: (0, k)),
            pl.BlockSpec((1, D_IN), lambda i, k: (0, 0)),
        ],
        out_specs=pl.BlockSpec((br, D_IN), lambda i, k: (i, 0)),
        out_shape=jax.ShapeDtypeStruct((B, D_IN), jnp.float32),
    )(latents, W_dec_bf16, b_dec.reshape(1, D_IN))


def kernel(x, W_enc, b_enc, W_dec, b_dec):
    pre = _encode(x.astype(jnp.bfloat16), W_enc.astype(jnp.bfloat16), b_enc)
    latents = _topk_mask(pre)
    x_hat = _decode(latents, W_dec.astype(jnp.bfloat16), b_dec)
    return (x_hat, latents)
